# baseline (speedup 1.0000x reference)
import functools

import jax
import jax.numpy as jnp
from jax import lax
from jax.experimental import pallas as pl
from jax.experimental.pallas import tpu as pltpu
from jax.experimental.pallas import tpu_sc as plsc

N = 10000
E = 640000
H = 128
DN = 128
DE = 16
G = 64

NC = 2
NS = 16
NW = NC * NS

GRP = 128
SUP = 1024
NSUP = E // SUP
NSUP_A = 312
NSUP_B = NSUP - NSUP_A
E_A = NSUP_A * SUP
E_B = NSUP_B * SUP
RSUB = 632
RLAST = N - 15 * RSUB

_mesh = plsc.VectorSubcoreMesh(core_axis_name="c", subcore_axis_name="s")


def _worker_id():
    return lax.axis_index("s") * NC + lax.axis_index("c")


def _chunk_base(wid, m):
    return (wid + (m // 8) * NW) * SUP + (m % 8) * GRP


def _prefetch_idx(idx_hbm, idx_v, wid, niter, isem, kmax):
    def wave(g, _):
        for u in range(8):
            @pl.when(g * 8 + u < niter)
            def _():
                k = g * 8 + u
                pltpu.async_copy(idx_hbm.at[wid + k * NW],
                                 idx_v.at[pl.ds(k * 8, 8)], isem)
        for u in range(8):
            @pl.when(g * 8 + u < niter)
            def _():
                k = g * 8 + u
                pltpu.make_async_copy(idx_hbm.at[wid + k * NW],
                                      idx_v.at[pl.ds(k * 8, 8)], isem).wait()
        return 0
    lax.fori_loop(0, (kmax + 7) // 8, wave, 0)


def _make_sc_gather(nsup):
    kmax = (nsup + NW - 1) // NW

    @functools.partial(
        pl.kernel,
        out_type=jax.ShapeDtypeStruct((nsup * SUP, H), jnp.float32),
        mesh=_mesh,
        scratch_types=[
            pltpu.VMEM((kmax * 8, GRP), jnp.int32),
            pltpu.VMEM((GRP, H), jnp.float32),
            pltpu.VMEM((GRP, H), jnp.float32),
            pltpu.VMEM((GRP, H), jnp.float32),
            pltpu.VMEM((GRP, H), jnp.float32),
            pltpu.SemaphoreType.DMA,
            pltpu.SemaphoreType.DMA,
            pltpu.SemaphoreType.DMA,
            pltpu.SemaphoreType.DMA,
            pltpu.SemaphoreType.DMA,
            pltpu.SemaphoreType.DMA,
            pltpu.SemaphoreType.DMA,
            pltpu.SemaphoreType.DMA,
            pltpu.SemaphoreType.DMA,
        ],
    )
    def gather_kernel(table_hbm, idx_hbm, out_hbm, idx_v, b0, b1, b2, b3,
                      g0, g1, g2, g3, o0, o1, o2, o3, isem):
        wid = _worker_id()
        niter = (nsup - wid + NW - 1) // NW
        M = 8 * niter
        buf = (b0, b1, b2, b3)
        gsem = (g0, g1, g2, g3)
        osem = (o0, o1, o2, o3)

        _prefetch_idx(idx_hbm, idx_v, wid, niter, isem, kmax)

        def fire_gather(m, i):
            pltpu.async_copy(table_hbm.at[idx_v.at[m]], buf[i], gsem[i])

        def wait_gather(m, i):
            pltpu.make_async_copy(table_hbm.at[idx_v.at[m]], buf[i],
                                  gsem[i]).wait()

        def fire_out(m, i):
            pltpu.async_copy(
                buf[i], out_hbm.at[pl.ds(_chunk_base(wid, m), GRP)], osem[i])

        def wait_out(m, i):
            pltpu.make_async_copy(
                buf[i], out_hbm.at[pl.ds(_chunk_base(wid, m), GRP)],
                osem[i]).wait()

        fire_gather(0, 0)
        fire_gather(1, 1)
        wait_gather(0, 0)
        wait_gather(1, 1)
        fire_out(0, 0)
        fire_out(1, 1)
        fire_gather(2, 2)
        fire_gather(3, 3)
        wait_gather(2, 2)
        wait_gather(3, 3)
        fire_out(2, 2)
        fire_out(3, 3)
        wait_out(0, 0)
        wait_out(1, 1)
        fire_gather(4, 0)
        fire_gather(5, 1)

        def body(p, _):
            m0 = 4 * p
            wait_gather(m0, 0)
            wait_gather(m0 + 1, 1)
            fire_out(m0, 0)
            fire_out(m0 + 1, 1)
            wait_out(m0 - 2, 2)
            wait_out(m0 - 1, 3)
            fire_gather(m0 + 2, 2)
            fire_gather(m0 + 3, 3)
            wait_gather(m0 + 2, 2)
            wait_gather(m0 + 3, 3)
            fire_out(m0 + 2, 2)
            fire_out(m0 + 3, 3)
            wait_out(m0, 0)
            wait_out(m0 + 1, 1)

            @pl.when(m0 + 4 < M)
            def _():
                fire_gather(m0 + 4, 0)
                fire_gather(m0 + 5, 1)

            return 0

        lax.fori_loop(1, M // 4, body, 0)
        wait_out(M - 2, 2)
        wait_out(M - 1, 3)

    return gather_kernel


def _make_sc_scatter(nsup):
    @functools.partial(
        pl.kernel,
        out_type=jax.ShapeDtypeStruct((NC, N, H), jnp.float32),
        mesh=_mesh,
        scratch_types=[
            pltpu.VMEM((8, GRP), jnp.int32),
            pltpu.VMEM((8, GRP), jnp.int32),
            pltpu.VMEM((GRP, H), jnp.float32),
            pltpu.VMEM((GRP, H), jnp.float32),
            pltpu.VMEM_SHARED((N, H), jnp.float32),
            pltpu.SemaphoreType.DMA,
            pltpu.SemaphoreType.DMA,
            pltpu.SemaphoreType.DMA,
            pltpu.SemaphoreType.DMA,
            pltpu.SemaphoreType.DMA,
            pltpu.SemaphoreType.DMA,
        ],
    )
    def scatter_kernel(g_hbm, idx_hbm, zero_hbm, out_hbm, iv0, iv1, b0, b1,
                       table_sh, d0, d1, s0, s1, i0, i1):
        c = lax.axis_index("c")
        s = lax.axis_index("s")
        wid = s * NC + c
        niter = (nsup - wid + NW - 1) // NW
        M = 8 * niter
        buf = (b0, b1)
        iv = (iv0, iv1)
        dsem = (d0, d1)
        ssem = (s0, s1)
        isem = (i0, i1)

        def fire_idx(k, pr):
            pltpu.async_copy(idx_hbm.at[wid + k * NW], iv[pr], isem[pr])

        def wait_idx(k, pr):
            pltpu.make_async_copy(idx_hbm.at[wid + k * NW], iv[pr],
                                  isem[pr]).wait()

        def fire_data(m, i):
            pltpu.async_copy(g_hbm.at[pl.ds(_chunk_base(wid, m), GRP)],
                             buf[i], dsem[i])

        def wait_data(m, i):
            pltpu.make_async_copy(
                g_hbm.at[pl.ds(_chunk_base(wid, m), GRP)], buf[i],
                dsem[i]).wait()

        def fire_scat(m, i):
            r = m % 8
            kp = (m // 8) % 2

            @pl.when(kp == 0)
            def _():
                pltpu.async_copy(buf[i], table_sh.at[iv0.at[r]], ssem[i],
                                 add=True)

            @pl.when(kp == 1)
            def _():
                pltpu.async_copy(buf[i], table_sh.at[iv1.at[r]], ssem[i],
                                 add=True)

        def wait_scat(m, i):
            r = m % 8
            kp = (m // 8) % 2

            @pl.when(kp == 0)
            def _():
                pltpu.make_async_copy(buf[i], table_sh.at[iv0.at[r]],
                                      ssem[i]).wait()

            @pl.when(kp == 1)
            def _():
                pltpu.make_async_copy(buf[i], table_sh.at[iv1.at[r]],
                                      ssem[i]).wait()

        fire_idx(0, 0)
        fire_idx(1, 1)

        @pl.when(s < NS - 1)
        def _():
            pltpu.sync_copy(zero_hbm.at[pl.ds(s * RSUB, RSUB)],
                            table_sh.at[pl.ds(s * RSUB, RSUB)])

        @pl.when(s == NS - 1)
        def _():
            pltpu.sync_copy(zero_hbm.at[pl.ds(15 * RSUB, RLAST)],
                            table_sh.at[pl.ds(15 * RSUB, RLAST)])

        plsc.subcore_barrier()

        wait_idx(0, 0)
        fire_data(0, 0)
        wait_data(0, 0)
        fire_scat(0, 0)
        fire_data(1, 1)
        wait_data(1, 1)
        fire_scat(1, 1)
        wait_scat(0, 0)
        fire_data(2, 0)

        def body(q, _):
            m0 = 2 * q
            k = m0 // 8

            @pl.when(m0 % 8 == 0)
            def _():
                @pl.when(k % 2 == 0)
                def _():
                    wait_idx(k, 0)

                @pl.when(k % 2 == 1)
                def _():
                    wait_idx(k, 1)

            wait_data(m0, 0)
            fire_scat(m0, 0)
            wait_scat(m0 - 1, 1)

            @pl.when(m0 % 8 == 0)
            def _():
                @pl.when(k + 1 < niter)
                def _():
                    @pl.when((k + 1) % 2 == 0)
                    def _():
                        fire_idx(k + 1, 0)

                    @pl.when((k + 1) % 2 == 1)
                    def _():
                        fire_idx(k + 1, 1)

            fire_data(m0 + 1, 1)
            wait_data(m0 + 1, 1)
            fire_scat(m0 + 1, 1)
            wait_scat(m0, 0)

            @pl.when(m0 + 2 < M)
            def _():
                fire_data(m0 + 2, 0)

            return 0

        lax.fori_loop(1, M // 2, body, 0)
        wait_scat(M - 1, 1)
        plsc.subcore_barrier()

        @pl.when(s < NS - 1)
        def _():
            pltpu.sync_copy(table_sh.at[pl.ds(s * RSUB, RSUB)],
                            out_hbm.at[c, pl.ds(s * RSUB, RSUB)])

        @pl.when(s == NS - 1)
        def _():
            pltpu.sync_copy(table_sh.at[pl.ds(15 * RSUB, RLAST)],
                            out_hbm.at[c, pl.ds(15 * RSUB, RLAST)])

    return scatter_kernel


_sc_gather_a = _make_sc_gather(NSUP_A)
_sc_gather_b = _make_sc_gather(NSUP_B)
_sc_scatter_a = _make_sc_scatter(NSUP_A)
_sc_scatter_b = _make_sc_scatter(NSUP_B)


def _dot(a, b):
    return jnp.dot(a, b, preferred_element_type=jnp.float32)


def _tc_xw_body(x_ref, w_ref, o_ref):
    o_ref[...] = _dot(x_ref[...], w_ref[...])


def _tc_xw(x, w):
    return pl.pallas_call(
        _tc_xw_body,
        out_shape=jax.ShapeDtypeStruct((N, H), jnp.float32),
    )(x, w)


def _tc_add4_body(pa_ref, pb_ref, o_ref):
    o_ref[...] = (pa_ref[0] + pa_ref[1]) + (pb_ref[0] + pb_ref[1])


def _tc_add4(pa, pb):
    return pl.pallas_call(
        _tc_add4_body,
        out_shape=jax.ShapeDtypeStruct((N, H), jnp.float32),
    )(pa, pb)


def _tc_edge_init_body(xr_ref, ea_ref, we_ref, bei_ref, w0_ref,
                       h0_ref, g0_ref):
    h0 = jax.nn.relu(xr_ref[...] + _dot(ea_ref[...], we_ref[...])
                     + bei_ref[...])
    h0_ref[...] = h0.astype(jnp.bfloat16)
    g0_ref[...] = _dot(h0, w0_ref[...])


def _make_tc_edge_init(nrows, blk):
    def run(xr, ea, we, bei, w0):
        return pl.pallas_call(
            _tc_edge_init_body,
            grid=(nrows // blk,),
            in_specs=[
                pl.BlockSpec((blk, H), lambda i: (i, 0)),
                pl.BlockSpec((blk, DE), lambda i: (i, 0)),
                pl.BlockSpec((DE, H), lambda i: (0, 0)),
                pl.BlockSpec((1, H), lambda i: (0, 0)),
                pl.BlockSpec((H, H), lambda i: (0, 0)),
            ],
            out_specs=[
                pl.BlockSpec((blk, H), lambda i: (i, 0)),
                pl.BlockSpec((blk, H), lambda i: (i, 0)),
            ],
            out_shape=[
                jax.ShapeDtypeStruct((nrows, H), jnp.bfloat16),
                jax.ShapeDtypeStruct((nrows, H), jnp.float32),
            ],
        )(xr, ea, we, bei, w0)
    return run


def _tc_conv_body(ar_ref, g_ref, h0_ref, w_ref, b_ref, o_ref):
    ge = g_ref[:, 0, :]
    go = g_ref[:, 1, :]
    h0e = h0_ref[:, 0, :].astype(jnp.float32)
    h0o = h0_ref[:, 1, :].astype(jnp.float32)
    he = jax.nn.relu(ar_ref[:, 0, :] - go + b_ref[...] + h0e)
    ho = jax.nn.relu(ar_ref[:, 1, :] - ge + b_ref[...] + h0o)
    o_ref[:, 0, :] = _dot(he, w_ref[...])
    o_ref[:, 1, :] = _dot(ho, w_ref[...])


def _tc_conv_last_body(ar_ref, g_ref, h0_ref, b_ref, o_ref):
    ge = g_ref[:, 0, :]
    go = g_ref[:, 1, :]
    h0e = h0_ref[:, 0, :].astype(jnp.float32)
    h0o = h0_ref[:, 1, :].astype(jnp.float32)
    o_ref[:, 0, :] = jax.nn.relu(ar_ref[:, 0, :] - go + b_ref[...] + h0e)
    o_ref[:, 1, :] = jax.nn.relu(ar_ref[:, 1, :] - ge + b_ref[...] + h0o)


def _make_tc_conv(npairs, blk):
    spec = pl.BlockSpec((blk, 2, H), lambda i: (i, 0, 0))

    def run(ar3, g3, h03, w, b):
        return pl.pallas_call(
            _tc_conv_body,
            grid=(npairs // blk,),
            in_specs=[
                spec, spec, spec,
                pl.BlockSpec((H, H), lambda i: (0, 0)),
                pl.BlockSpec((1, H), lambda i: (0, 0)),
            ],
            out_specs=spec,
            out_shape=jax.ShapeDtypeStruct((npairs, 2, H), jnp.float32),
        )(ar3, g3, h03, w, b)
    return run


def _make_tc_conv_last(npairs, blk):
    spec = pl.BlockSpec((blk, 2, H), lambda i: (i, 0, 0))

    def run(ar3, g3, h03, b):
        return pl.pallas_call(
            _tc_conv_last_body,
            grid=(npairs // blk,),
            in_specs=[
                spec, spec, spec,
                pl.BlockSpec((1, H), lambda i: (0, 0)),
            ],
            out_specs=spec,
            out_shape=jax.ShapeDtypeStruct((npairs, 2, H), jnp.float32),
        )(ar3, g3, h03, b)
    return run


_tc_edge_init_a = _make_tc_edge_init(E_A, 4992)
_tc_edge_init_b = _make_tc_edge_init(E_B, 5008)
_tc_conv_a = _make_tc_conv(E_A // 2, 2496)
_tc_conv_b = _make_tc_conv(E_B // 2, 2504)
_tc_conv_last_a = _make_tc_conv_last(E_A // 2, 2496)
_tc_conv_last_b = _make_tc_conv_last(E_B // 2, 2504)

_BN = 1000


def _tc_final_body(x_ref, s_ref, b2_ref, w1_ref, w2_ref, be_ref, wf_ref,
                   bf_ref, o_ref, acc_ref):
    i = pl.program_id(0)

    @pl.when(i == 0)
    def _():
        acc_ref[...] = jnp.zeros_like(acc_ref)

    hn = jax.nn.relu(_dot(x_ref[...], w1_ref[...])
                     + _dot(s_ref[...], w2_ref[...]) + be_ref[...])
    onehot = (b2_ref[...] == lax.broadcasted_iota(jnp.int32, (_BN, G), 1))
    acc_ref[...] += lax.dot_general(
        onehot.astype(jnp.float32), hn,
        (((0,), (0,)), ((), ())), preferred_element_type=jnp.float32)

    @pl.when(i == pl.num_programs(0) - 1)
    def _():
        o_ref[...] = (jnp.sum(acc_ref[...] * wf_ref[...], axis=1,
                              keepdims=True) + bf_ref[...])


def _tc_final(x, s, batch2, w1, w2, be, wf_row, bf):
    return pl.pallas_call(
        _tc_final_body,
        grid=(N // _BN,),
        in_specs=[
            pl.BlockSpec((_BN, DN), lambda i: (i, 0)),
            pl.BlockSpec((_BN, H), lambda i: (i, 0)),
            pl.BlockSpec((_BN, 1), lambda i: (i, 0)),
            pl.BlockSpec((DN, H), lambda i: (0, 0)),
            pl.BlockSpec((H, H), lambda i: (0, 0)),
            pl.BlockSpec((1, H), lambda i: (0, 0)),
            pl.BlockSpec((1, H), lambda i: (0, 0)),
            pl.BlockSpec((1, 1), lambda i: (0, 0)),
        ],
        out_specs=pl.BlockSpec((G, 1), lambda i: (0, 0)),
        out_shape=jax.ShapeDtypeStruct((G, 1), jnp.float32),
        scratch_shapes=[pltpu.VMEM((G, H), jnp.float32)],
    )(x, s, batch2, w1, w2, be, wf_row, bf)


def kernel(x, edge_attr, W_edge_init, b_edge_init, W_conv0, b_conv0,
           W_conv1, b_conv1, W_conv2, b_conv2, W_e2n, b_e2n, W_ffn, b_ffn,
           edge_index, batch):
    row = edge_index[0].astype(jnp.int32)
    col = edge_index[1].astype(jnp.int32)
    row3 = row.reshape(NSUP, 8, GRP)
    col3 = col.reshape(NSUP, 8, GRP)
    rowA, rowB = row3[:NSUP_A], row3[NSUP_A:]
    colA, colB = col3[:NSUP_A], col3[NSUP_A:]
    eaA, eaB = edge_attr[:E_A], edge_attr[E_A:]
    zeros_n = jnp.zeros((N, H), jnp.float32)

    bei = b_edge_init.reshape(1, H)
    b0 = b_conv0.reshape(1, H)
    b1 = b_conv1.reshape(1, H)
    b2 = b_conv2.reshape(1, H)
    be = b_e2n.reshape(1, H)
    wf_row = W_ffn.reshape(1, H)
    bf = b_ffn.reshape(1, 1)
    batch2 = batch.astype(jnp.int32).reshape(N, 1)

    xw = _tc_xw(x, W_edge_init[:DN])
    xrA = _sc_gather_a(xw, rowA)
    xrB = _sc_gather_b(xw, rowB)
    h0A, g0A = _tc_edge_init_a(xrA, eaA, W_edge_init[DN:], bei, W_conv0)
    h0B, g0B = _tc_edge_init_b(xrB, eaB, W_edge_init[DN:], bei, W_conv0)

    h0A3 = h0A.reshape(E_A // 2, 2, H)
    h0B3 = h0B.reshape(E_B // 2, 2, H)
    gA, gB = g0A, g0B
    for w_next, b_cur in ((W_conv1, b0), (W_conv2, b1)):
        pA = _sc_scatter_a(gA, colA, zeros_n)
        pB = _sc_scatter_b(gB, colB, zeros_n)
        a = _tc_add4(pA, pB)
        arA3 = _sc_gather_a(a, rowA).reshape(E_A // 2, 2, H)
        gA = _tc_conv_a(arA3, gA.reshape(E_A // 2, 2, H), h0A3,
                        w_next, b_cur).reshape(E_A, H)
        arB3 = _sc_gather_b(a, rowB).reshape(E_B // 2, 2, H)
        gB = _tc_conv_b(arB3, gB.reshape(E_B // 2, 2, H), h0B3,
                        w_next, b_cur).reshape(E_B, H)

    pA = _sc_scatter_a(gA, colA, zeros_n)
    pB = _sc_scatter_b(gB, colB, zeros_n)
    a = _tc_add4(pA, pB)
    arA3 = _sc_gather_a(a, rowA).reshape(E_A // 2, 2, H)
    h3A = _tc_conv_last_a(arA3, gA.reshape(E_A // 2, 2, H), h0A3, b2)
    arB3 = _sc_gather_b(a, rowB).reshape(E_B // 2, 2, H)
    h3B = _tc_conv_last_b(arB3, gB.reshape(E_B // 2, 2, H), h0B3, b2)

    pA = _sc_scatter_a(h3A.reshape(E_A, H), colA, zeros_n)
    pB = _sc_scatter_b(h3B.reshape(E_B, H), colB, zeros_n)
    s = _tc_add4(pA, pB)
    out = _tc_final(x, s, batch2, W_e2n[:DN], W_e2n[DN:], be, wf_row, bf)
    return out.reshape(G)


# speedup vs baseline: 1.0062x; 1.0062x over previous
import functools

import jax
import jax.numpy as jnp
from jax import lax
from jax.experimental import pallas as pl
from jax.experimental.pallas import tpu as pltpu
from jax.experimental.pallas import tpu_sc as plsc

N = 10000
E = 640000
H = 128
DN = 128
DE = 16
G = 64

NC = 2
NS = 16
NW = NC * NS

GRP = 128
SUP = 1024
NSUP = E // SUP
NSUP_A = 312
NSUP_B = NSUP - NSUP_A
E_A = NSUP_A * SUP
E_B = NSUP_B * SUP
RSUB = 632
RLAST = N - 15 * RSUB

_mesh = plsc.VectorSubcoreMesh(core_axis_name="c", subcore_axis_name="s")


def _worker_id():
    return lax.axis_index("s") * NC + lax.axis_index("c")


def _chunk_base(wid, m):
    return (wid + (m // 8) * NW) * SUP + (m % 8) * GRP


def _prefetch_idx(idx_hbm, idx_v, wid, niter, isem, kmax):
    def wave(g, _):
        for u in range(8):
            @pl.when(g * 8 + u < niter)
            def _():
                k = g * 8 + u
                pltpu.async_copy(idx_hbm.at[wid + k * NW],
                                 idx_v.at[pl.ds(k * 8, 8)], isem)
        for u in range(8):
            @pl.when(g * 8 + u < niter)
            def _():
                k = g * 8 + u
                pltpu.make_async_copy(idx_hbm.at[wid + k * NW],
                                      idx_v.at[pl.ds(k * 8, 8)], isem).wait()
        return 0
    lax.fori_loop(0, (kmax + 7) // 8, wave, 0)


def _make_sc_gather(nsup):
    kmax = (nsup + NW - 1) // NW

    @functools.partial(
        pl.kernel,
        out_type=jax.ShapeDtypeStruct((nsup * SUP, H), jnp.float32),
        mesh=_mesh,
        scratch_types=[
            pltpu.VMEM((kmax * 8, GRP), jnp.int32),
            pltpu.VMEM((GRP, H), jnp.float32),
            pltpu.VMEM((GRP, H), jnp.float32),
            pltpu.VMEM((GRP, H), jnp.float32),
            pltpu.VMEM((GRP, H), jnp.float32),
            pltpu.SemaphoreType.DMA,
            pltpu.SemaphoreType.DMA,
            pltpu.SemaphoreType.DMA,
            pltpu.SemaphoreType.DMA,
            pltpu.SemaphoreType.DMA,
            pltpu.SemaphoreType.DMA,
            pltpu.SemaphoreType.DMA,
            pltpu.SemaphoreType.DMA,
            pltpu.SemaphoreType.DMA,
        ],
    )
    def gather_kernel(table_hbm, idx_hbm, out_hbm, idx_v, b0, b1, b2, b3,
                      g0, g1, g2, g3, o0, o1, o2, o3, isem):
        wid = _worker_id()
        niter = (nsup - wid + NW - 1) // NW
        M = 8 * niter
        buf = (b0, b1, b2, b3)
        gsem = (g0, g1, g2, g3)
        osem = (o0, o1, o2, o3)

        _prefetch_idx(idx_hbm, idx_v, wid, niter, isem, kmax)

        def fire_gather(m, i):
            pltpu.async_copy(table_hbm.at[idx_v.at[m]], buf[i], gsem[i])

        def wait_gather(m, i):
            pltpu.make_async_copy(table_hbm.at[idx_v.at[m]], buf[i],
                                  gsem[i]).wait()

        def fire_out(m, i):
            pltpu.async_copy(
                buf[i], out_hbm.at[pl.ds(_chunk_base(wid, m), GRP)], osem[i])

        def wait_out(m, i):
            pltpu.make_async_copy(
                buf[i], out_hbm.at[pl.ds(_chunk_base(wid, m), GRP)],
                osem[i]).wait()

        fire_gather(0, 0)
        fire_gather(1, 1)
        wait_gather(0, 0)
        wait_gather(1, 1)
        fire_out(0, 0)
        fire_out(1, 1)
        fire_gather(2, 2)
        fire_gather(3, 3)
        wait_gather(2, 2)
        wait_gather(3, 3)
        fire_out(2, 2)
        fire_out(3, 3)
        wait_out(0, 0)
        wait_out(1, 1)
        fire_gather(4, 0)
        fire_gather(5, 1)

        def body(p, _):
            m0 = 4 * p
            wait_gather(m0, 0)
            wait_gather(m0 + 1, 1)
            fire_out(m0, 0)
            fire_out(m0 + 1, 1)
            wait_out(m0 - 2, 2)
            wait_out(m0 - 1, 3)
            fire_gather(m0 + 2, 2)
            fire_gather(m0 + 3, 3)
            wait_gather(m0 + 2, 2)
            wait_gather(m0 + 3, 3)
            fire_out(m0 + 2, 2)
            fire_out(m0 + 3, 3)
            wait_out(m0, 0)
            wait_out(m0 + 1, 1)

            @pl.when(m0 + 4 < M)
            def _():
                fire_gather(m0 + 4, 0)
                fire_gather(m0 + 5, 1)

            return 0

        lax.fori_loop(1, M // 4, body, 0)
        wait_out(M - 2, 2)
        wait_out(M - 1, 3)

    return gather_kernel


SGRP = 64


def _scat_base(wid, m):
    return (wid + (m // 16) * NW) * SUP + (m % 16) * SGRP


def _make_sc_scatter(nsup):
    @functools.partial(
        pl.kernel,
        out_type=jax.ShapeDtypeStruct((NC, N, H), jnp.float32),
        mesh=_mesh,
        scratch_types=[
            pltpu.VMEM((16, SGRP), jnp.int32),
            pltpu.VMEM((16, SGRP), jnp.int32),
            pltpu.VMEM((SGRP, H), jnp.float32),
            pltpu.VMEM((SGRP, H), jnp.float32),
            pltpu.VMEM((SGRP, H), jnp.float32),
            pltpu.VMEM((SGRP, H), jnp.float32),
            pltpu.VMEM_SHARED((N, H), jnp.float32),
            pltpu.SemaphoreType.DMA,
            pltpu.SemaphoreType.DMA,
            pltpu.SemaphoreType.DMA,
            pltpu.SemaphoreType.DMA,
            pltpu.SemaphoreType.DMA,
            pltpu.SemaphoreType.DMA,
            pltpu.SemaphoreType.DMA,
            pltpu.SemaphoreType.DMA,
            pltpu.SemaphoreType.DMA,
            pltpu.SemaphoreType.DMA,
        ],
    )
    def scatter_kernel(g_hbm, idx_hbm, zero_hbm, out_hbm, iv0, iv1,
                       b0, b1, b2, b3, table_sh,
                       d0, d1, d2, d3, s0, s1, s2, s3, i0, i1):
        c = lax.axis_index("c")
        s = lax.axis_index("s")
        wid = s * NC + c
        niter = (nsup - wid + NW - 1) // NW
        M = 16 * niter
        buf = (b0, b1, b2, b3)
        iv = (iv0, iv1)
        dsem = (d0, d1, d2, d3)
        ssem = (s0, s1, s2, s3)
        isem = (i0, i1)

        def fire_idx(k, pr):
            pltpu.async_copy(idx_hbm.at[wid + k * NW], iv[pr], isem[pr])

        def wait_idx(k, pr):
            pltpu.make_async_copy(idx_hbm.at[wid + k * NW], iv[pr],
                                  isem[pr]).wait()

        def idx_boundary(m0):
            k = m0 // 16

            @pl.when(m0 % 16 == 0)
            def _():
                @pl.when(k % 2 == 0)
                def _():
                    wait_idx(k, 0)

                @pl.when(k % 2 == 1)
                def _():
                    wait_idx(k, 1)

        def idx_refire(m0):
            k = m0 // 16

            @pl.when(m0 % 16 == 0)
            def _():
                @pl.when(k + 1 < niter)
                def _():
                    @pl.when((k + 1) % 2 == 0)
                    def _():
                        fire_idx(k + 1, 0)

                    @pl.when((k + 1) % 2 == 1)
                    def _():
                        fire_idx(k + 1, 1)

        def fire_data(m, i):
            pltpu.async_copy(g_hbm.at[pl.ds(_scat_base(wid, m), SGRP)],
                             buf[i], dsem[i])

        def wait_data(m, i):
            pltpu.make_async_copy(
                g_hbm.at[pl.ds(_scat_base(wid, m), SGRP)], buf[i],
                dsem[i]).wait()

        def fire_scat(m, i):
            r = m % 16
            kp = (m // 16) % 2

            @pl.when(kp == 0)
            def _():
                pltpu.async_copy(buf[i], table_sh.at[iv0.at[r]], ssem[i],
                                 add=True)

            @pl.when(kp == 1)
            def _():
                pltpu.async_copy(buf[i], table_sh.at[iv1.at[r]], ssem[i],
                                 add=True)

        def wait_scat(m, i):
            r = m % 16
            kp = (m // 16) % 2

            @pl.when(kp == 0)
            def _():
                pltpu.make_async_copy(buf[i], table_sh.at[iv0.at[r]],
                                      ssem[i]).wait()

            @pl.when(kp == 1)
            def _():
                pltpu.make_async_copy(buf[i], table_sh.at[iv1.at[r]],
                                      ssem[i]).wait()

        fire_idx(0, 0)
        fire_idx(1, 1)

        @pl.when(s < NS - 1)
        def _():
            pltpu.sync_copy(zero_hbm.at[pl.ds(s * RSUB, RSUB)],
                            table_sh.at[pl.ds(s * RSUB, RSUB)])

        @pl.when(s == NS - 1)
        def _():
            pltpu.sync_copy(zero_hbm.at[pl.ds(15 * RSUB, RLAST)],
                            table_sh.at[pl.ds(15 * RSUB, RLAST)])

        plsc.subcore_barrier()

        wait_idx(0, 0)
        fire_data(0, 0)
        fire_data(1, 1)
        wait_data(0, 0)
        wait_data(1, 1)
        fire_scat(0, 0)
        fire_scat(1, 1)
        fire_data(2, 2)
        fire_data(3, 3)
        wait_data(2, 2)
        wait_data(3, 3)
        fire_scat(2, 2)
        fire_scat(3, 3)
        wait_scat(0, 0)
        wait_scat(1, 1)
        fire_data(4, 0)
        fire_data(5, 1)

        def body(p, _):
            m0 = 4 * p
            idx_boundary(m0)
            wait_data(m0, 0)
            wait_data(m0 + 1, 1)
            fire_scat(m0, 0)
            fire_scat(m0 + 1, 1)
            wait_scat(m0 - 2, 2)
            wait_scat(m0 - 1, 3)
            idx_refire(m0)
            fire_data(m0 + 2, 2)
            fire_data(m0 + 3, 3)
            wait_data(m0 + 2, 2)
            wait_data(m0 + 3, 3)
            fire_scat(m0 + 2, 2)
            fire_scat(m0 + 3, 3)
            wait_scat(m0, 0)
            wait_scat(m0 + 1, 1)

            @pl.when(m0 + 4 < M)
            def _():
                fire_data(m0 + 4, 0)
                fire_data(m0 + 5, 1)

            return 0

        lax.fori_loop(1, M // 4, body, 0)
        wait_scat(M - 2, 2)
        wait_scat(M - 1, 3)
        plsc.subcore_barrier()

        @pl.when(s < NS - 1)
        def _():
            pltpu.sync_copy(table_sh.at[pl.ds(s * RSUB, RSUB)],
                            out_hbm.at[c, pl.ds(s * RSUB, RSUB)])

        @pl.when(s == NS - 1)
        def _():
            pltpu.sync_copy(table_sh.at[pl.ds(15 * RSUB, RLAST)],
                            out_hbm.at[c, pl.ds(15 * RSUB, RLAST)])

    return scatter_kernel


_sc_gather_a = _make_sc_gather(NSUP_A)
_sc_gather_b = _make_sc_gather(NSUP_B)
_sc_scatter_a = _make_sc_scatter(NSUP_A)
_sc_scatter_b = _make_sc_scatter(NSUP_B)


def _dot(a, b):
    return jnp.dot(a, b, preferred_element_type=jnp.float32)


def _tc_xw_body(x_ref, w_ref, o_ref):
    o_ref[...] = _dot(x_ref[...], w_ref[...])


def _tc_xw(x, w):
    return pl.pallas_call(
        _tc_xw_body,
        out_shape=jax.ShapeDtypeStruct((N, H), jnp.float32),
    )(x, w)


def _tc_add4_body(pa_ref, pb_ref, o_ref):
    o_ref[...] = (pa_ref[0] + pa_ref[1]) + (pb_ref[0] + pb_ref[1])


def _tc_add4(pa, pb):
    return pl.pallas_call(
        _tc_add4_body,
        out_shape=jax.ShapeDtypeStruct((N, H), jnp.float32),
    )(pa, pb)


def _tc_edge_init_body(xr_ref, ea_ref, we_ref, bei_ref, w0_ref,
                       h0_ref, g0_ref):
    h0 = jax.nn.relu(xr_ref[...] + _dot(ea_ref[...], we_ref[...])
                     + bei_ref[...])
    h0_ref[...] = h0
    g0_ref[...] = _dot(h0, w0_ref[...])


def _make_tc_edge_init(nrows, blk):
    def run(xr, ea, we, bei, w0):
        return pl.pallas_call(
            _tc_edge_init_body,
            grid=(nrows // blk,),
            in_specs=[
                pl.BlockSpec((blk, H), lambda i: (i, 0)),
                pl.BlockSpec((blk, DE), lambda i: (i, 0)),
                pl.BlockSpec((DE, H), lambda i: (0, 0)),
                pl.BlockSpec((1, H), lambda i: (0, 0)),
                pl.BlockSpec((H, H), lambda i: (0, 0)),
            ],
            out_specs=[
                pl.BlockSpec((blk, H), lambda i: (i, 0)),
                pl.BlockSpec((blk, H), lambda i: (i, 0)),
            ],
            out_shape=[
                jax.ShapeDtypeStruct((nrows, H), jnp.float32),
                jax.ShapeDtypeStruct((nrows, H), jnp.float32),
            ],
        )(xr, ea, we, bei, w0)
    return run


def _tc_conv_body(ar_ref, g_ref, h0_ref, w_ref, b_ref, o_ref):
    ge = g_ref[:, 0, :]
    go = g_ref[:, 1, :]
    he = jax.nn.relu(ar_ref[:, 0, :] - go + b_ref[...] + h0_ref[:, 0, :])
    ho = jax.nn.relu(ar_ref[:, 1, :] - ge + b_ref[...] + h0_ref[:, 1, :])
    o_ref[:, 0, :] = _dot(he, w_ref[...])
    o_ref[:, 1, :] = _dot(ho, w_ref[...])


def _tc_conv_last_body(ar_ref, g_ref, h0_ref, b_ref, o_ref):
    ge = g_ref[:, 0, :]
    go = g_ref[:, 1, :]
    o_ref[:, 0, :] = jax.nn.relu(ar_ref[:, 0, :] - go + b_ref[...]
                                 + h0_ref[:, 0, :])
    o_ref[:, 1, :] = jax.nn.relu(ar_ref[:, 1, :] - ge + b_ref[...]
                                 + h0_ref[:, 1, :])


def _make_tc_conv(npairs, blk):
    spec = pl.BlockSpec((blk, 2, H), lambda i: (i, 0, 0))

    def run(ar3, g3, h03, w, b):
        return pl.pallas_call(
            _tc_conv_body,
            grid=(npairs // blk,),
            in_specs=[
                spec, spec, spec,
                pl.BlockSpec((H, H), lambda i: (0, 0)),
                pl.BlockSpec((1, H), lambda i: (0, 0)),
            ],
            out_specs=spec,
            out_shape=jax.ShapeDtypeStruct((npairs, 2, H), jnp.float32),
        )(ar3, g3, h03, w, b)
    return run


def _make_tc_conv_last(npairs, blk):
    spec = pl.BlockSpec((blk, 2, H), lambda i: (i, 0, 0))

    def run(ar3, g3, h03, b):
        return pl.pallas_call(
            _tc_conv_last_body,
            grid=(npairs // blk,),
            in_specs=[
                spec, spec, spec,
                pl.BlockSpec((1, H), lambda i: (0, 0)),
            ],
            out_specs=spec,
            out_shape=jax.ShapeDtypeStruct((npairs, 2, H), jnp.float32),
        )(ar3, g3, h03, b)
    return run


_tc_edge_init_a = _make_tc_edge_init(E_A, 4992)
_tc_edge_init_b = _make_tc_edge_init(E_B, 5008)
_tc_conv_a = _make_tc_conv(E_A // 2, 2496)
_tc_conv_b = _make_tc_conv(E_B // 2, 2504)
_tc_conv_last_a = _make_tc_conv_last(E_A // 2, 2496)
_tc_conv_last_b = _make_tc_conv_last(E_B // 2, 2504)

_BN = 1000


def _tc_final_body(x_ref, s_ref, b2_ref, w1_ref, w2_ref, be_ref, wf_ref,
                   bf_ref, o_ref, acc_ref):
    i = pl.program_id(0)

    @pl.when(i == 0)
    def _():
        acc_ref[...] = jnp.zeros_like(acc_ref)

    hn = jax.nn.relu(_dot(x_ref[...], w1_ref[...])
                     + _dot(s_ref[...], w2_ref[...]) + be_ref[...])
    onehot = (b2_ref[...] == lax.broadcasted_iota(jnp.int32, (_BN, G), 1))
    acc_ref[...] += lax.dot_general(
        onehot.astype(jnp.float32), hn,
        (((0,), (0,)), ((), ())), preferred_element_type=jnp.float32)

    @pl.when(i == pl.num_programs(0) - 1)
    def _():
        o_ref[...] = (jnp.sum(acc_ref[...] * wf_ref[...], axis=1,
                              keepdims=True) + bf_ref[...])


def _tc_final(x, s, batch2, w1, w2, be, wf_row, bf):
    return pl.pallas_call(
        _tc_final_body,
        grid=(N // _BN,),
        in_specs=[
            pl.BlockSpec((_BN, DN), lambda i: (i, 0)),
            pl.BlockSpec((_BN, H), lambda i: (i, 0)),
            pl.BlockSpec((_BN, 1), lambda i: (i, 0)),
            pl.BlockSpec((DN, H), lambda i: (0, 0)),
            pl.BlockSpec((H, H), lambda i: (0, 0)),
            pl.BlockSpec((1, H), lambda i: (0, 0)),
            pl.BlockSpec((1, H), lambda i: (0, 0)),
            pl.BlockSpec((1, 1), lambda i: (0, 0)),
        ],
        out_specs=pl.BlockSpec((G, 1), lambda i: (0, 0)),
        out_shape=jax.ShapeDtypeStruct((G, 1), jnp.float32),
        scratch_shapes=[pltpu.VMEM((G, H), jnp.float32)],
    )(x, s, batch2, w1, w2, be, wf_row, bf)


def kernel(x, edge_attr, W_edge_init, b_edge_init, W_conv0, b_conv0,
           W_conv1, b_conv1, W_conv2, b_conv2, W_e2n, b_e2n, W_ffn, b_ffn,
           edge_index, batch):
    row = edge_index[0].astype(jnp.int32)
    col = edge_index[1].astype(jnp.int32)
    row3 = row.reshape(NSUP, 8, GRP)
    rowA, rowB = row3[:NSUP_A], row3[NSUP_A:]
    col4 = col.reshape(NSUP, 16, SGRP)
    colA, colB = col4[:NSUP_A], col4[NSUP_A:]
    eaA, eaB = edge_attr[:E_A], edge_attr[E_A:]
    zeros_n = jnp.zeros((N, H), jnp.float32)

    bei = b_edge_init.reshape(1, H)
    b0 = b_conv0.reshape(1, H)
    b1 = b_conv1.reshape(1, H)
    b2 = b_conv2.reshape(1, H)
    be = b_e2n.reshape(1, H)
    wf_row = W_ffn.reshape(1, H)
    bf = b_ffn.reshape(1, 1)
    batch2 = batch.astype(jnp.int32).reshape(N, 1)

    xw = _tc_xw(x, W_edge_init[:DN])
    xrA = _sc_gather_a(xw, rowA)
    xrB = _sc_gather_b(xw, rowB)
    h0A, g0A = _tc_edge_init_a(xrA, eaA, W_edge_init[DN:], bei, W_conv0)
    h0B, g0B = _tc_edge_init_b(xrB, eaB, W_edge_init[DN:], bei, W_conv0)

    h0A3 = h0A.reshape(E_A // 2, 2, H)
    h0B3 = h0B.reshape(E_B // 2, 2, H)
    gA, gB = g0A, g0B
    for w_next, b_cur in ((W_conv1, b0), (W_conv2, b1)):
        pA = _sc_scatter_a(gA, colA, zeros_n)
        pB = _sc_scatter_b(gB, colB, zeros_n)
        a = _tc_add4(pA, pB)
        arA3 = _sc_gather_a(a, rowA).reshape(E_A // 2, 2, H)
        gA = _tc_conv_a(arA3, gA.reshape(E_A // 2, 2, H), h0A3,
                        w_next, b_cur).reshape(E_A, H)
        arB3 = _sc_gather_b(a, rowB).reshape(E_B // 2, 2, H)
        gB = _tc_conv_b(arB3, gB.reshape(E_B // 2, 2, H), h0B3,
                        w_next, b_cur).reshape(E_B, H)

    pA = _sc_scatter_a(gA, colA, zeros_n)
    pB = _sc_scatter_b(gB, colB, zeros_n)
    a = _tc_add4(pA, pB)
    arA3 = _sc_gather_a(a, rowA).reshape(E_A // 2, 2, H)
    h3A = _tc_conv_last_a(arA3, gA.reshape(E_A // 2, 2, H), h0A3, b2)
    arB3 = _sc_gather_b(a, rowB).reshape(E_B // 2, 2, H)
    h3B = _tc_conv_last_b(arB3, gB.reshape(E_B // 2, 2, H), h0B3, b2)

    pA = _sc_scatter_a(h3A.reshape(E_A, H), colA, zeros_n)
    pB = _sc_scatter_b(h3B.reshape(E_B, H), colB, zeros_n)
    s = _tc_add4(pA, pB)
    out = _tc_final(x, s, batch2, W_e2n[:DN], W_e2n[DN:], be, wf_row, bf)
    return out.reshape(G)


# speedup vs baseline: 1.0165x; 1.0102x over previous
import functools

import jax
import jax.numpy as jnp
from jax import lax
from jax.experimental import pallas as pl
from jax.experimental.pallas import tpu as pltpu
from jax.experimental.pallas import tpu_sc as plsc

N = 10000
E = 640000
H = 128
DN = 128
DE = 16
G = 64

NC = 2
NS = 16
NW = NC * NS

GRP = 128
SUP = 1024
NSUP = E // SUP
NSUP_A = 312
NSUP_B = NSUP - NSUP_A
E_A = NSUP_A * SUP
E_B = NSUP_B * SUP
RSUB = 632
RLAST = N - 15 * RSUB

_mesh = plsc.VectorSubcoreMesh(core_axis_name="c", subcore_axis_name="s")


def _worker_id():
    return lax.axis_index("s") * NC + lax.axis_index("c")


def _chunk_base(wid, m):
    return (wid + (m // 8) * NW) * SUP + (m % 8) * GRP


def _prefetch_idx(idx_hbm, idx_v, wid, niter, isem, kmax):
    def wave(g, _):
        for u in range(8):
            @pl.when(g * 8 + u < niter)
            def _():
                k = g * 8 + u
                pltpu.async_copy(idx_hbm.at[wid + k * NW],
                                 idx_v.at[pl.ds(k * 8, 8)], isem)
        for u in range(8):
            @pl.when(g * 8 + u < niter)
            def _():
                k = g * 8 + u
                pltpu.make_async_copy(idx_hbm.at[wid + k * NW],
                                      idx_v.at[pl.ds(k * 8, 8)], isem).wait()
        return 0
    lax.fori_loop(0, (kmax + 7) // 8, wave, 0)


def _make_sc_gather(nsup):
    kmax = (nsup + NW - 1) // NW

    @functools.partial(
        pl.kernel,
        out_type=jax.ShapeDtypeStruct((nsup * SUP, H), jnp.float32),
        mesh=_mesh,
        scratch_types=[
            pltpu.VMEM((kmax * 8, GRP), jnp.int32),
            pltpu.VMEM((GRP, H), jnp.float32),
            pltpu.VMEM((GRP, H), jnp.float32),
            pltpu.VMEM((GRP, H), jnp.float32),
            pltpu.VMEM((GRP, H), jnp.float32),
            pltpu.SemaphoreType.DMA,
            pltpu.SemaphoreType.DMA,
            pltpu.SemaphoreType.DMA,
            pltpu.SemaphoreType.DMA,
            pltpu.SemaphoreType.DMA,
            pltpu.SemaphoreType.DMA,
            pltpu.SemaphoreType.DMA,
            pltpu.SemaphoreType.DMA,
            pltpu.SemaphoreType.DMA,
        ],
    )
    def gather_kernel(table_hbm, idx_hbm, out_hbm, idx_v, b0, b1, b2, b3,
                      g0, g1, g2, g3, o0, o1, o2, o3, isem):
        wid = _worker_id()
        niter = (nsup - wid + NW - 1) // NW
        M = 8 * niter
        buf = (b0, b1, b2, b3)
        gsem = (g0, g1, g2, g3)
        osem = (o0, o1, o2, o3)

        _prefetch_idx(idx_hbm, idx_v, wid, niter, isem, kmax)

        def fire_gather(m, i):
            pltpu.async_copy(table_hbm.at[idx_v.at[m]], buf[i], gsem[i])

        def wait_gather(m, i):
            pltpu.make_async_copy(table_hbm.at[idx_v.at[m]], buf[i],
                                  gsem[i]).wait()

        def fire_out(m, i):
            pltpu.async_copy(
                buf[i], out_hbm.at[pl.ds(_chunk_base(wid, m), GRP)], osem[i])

        def wait_out(m, i):
            pltpu.make_async_copy(
                buf[i], out_hbm.at[pl.ds(_chunk_base(wid, m), GRP)],
                osem[i]).wait()

        fire_gather(0, 0)
        fire_gather(1, 1)
        wait_gather(0, 0)
        wait_gather(1, 1)
        fire_out(0, 0)
        fire_out(1, 1)
        fire_gather(2, 2)
        fire_gather(3, 3)
        wait_gather(2, 2)
        wait_gather(3, 3)
        fire_out(2, 2)
        fire_out(3, 3)
        wait_out(0, 0)
        wait_out(1, 1)
        fire_gather(4, 0)
        fire_gather(5, 1)

        def body(p, _):
            m0 = 4 * p
            wait_gather(m0, 0)
            wait_gather(m0 + 1, 1)
            fire_out(m0, 0)
            fire_out(m0 + 1, 1)
            wait_out(m0 - 2, 2)
            wait_out(m0 - 1, 3)
            fire_gather(m0 + 2, 2)
            fire_gather(m0 + 3, 3)
            wait_gather(m0 + 2, 2)
            wait_gather(m0 + 3, 3)
            fire_out(m0 + 2, 2)
            fire_out(m0 + 3, 3)
            wait_out(m0, 0)
            wait_out(m0 + 1, 1)

            @pl.when(m0 + 4 < M)
            def _():
                fire_gather(m0 + 4, 0)
                fire_gather(m0 + 5, 1)

            return 0

        lax.fori_loop(1, M // 4, body, 0)
        wait_out(M - 2, 2)
        wait_out(M - 1, 3)

    return gather_kernel


def _make_sc_scatter(nsup):
    @functools.partial(
        pl.kernel,
        out_type=jax.ShapeDtypeStruct((NC, N, H), jnp.float32),
        mesh=_mesh,
        scratch_types=[
            pltpu.VMEM((8, GRP), jnp.int32),
            pltpu.VMEM((8, GRP), jnp.int32),
            pltpu.VMEM((GRP, H), jnp.float32),
            pltpu.VMEM((GRP, H), jnp.float32),
            pltpu.VMEM_SHARED((N, H), jnp.float32),
            pltpu.SemaphoreType.DMA,
            pltpu.SemaphoreType.DMA,
            pltpu.SemaphoreType.DMA,
            pltpu.SemaphoreType.DMA,
            pltpu.SemaphoreType.DMA,
            pltpu.SemaphoreType.DMA,
        ],
    )
    def scatter_kernel(g_hbm, idx_hbm, zero_hbm, out_hbm, iv0, iv1, b0, b1,
                       table_sh, d0, d1, s0, s1, i0, i1):
        c = lax.axis_index("c")
        s = lax.axis_index("s")
        wid = s * NC + c
        niter = (nsup - wid + NW - 1) // NW
        M = 8 * niter
        buf = (b0, b1)
        iv = (iv0, iv1)
        dsem = (d0, d1)
        ssem = (s0, s1)
        isem = (i0, i1)

        def fire_idx(k, pr):
            pltpu.async_copy(idx_hbm.at[wid + k * NW], iv[pr], isem[pr])

        def wait_idx(k, pr):
            pltpu.make_async_copy(idx_hbm.at[wid + k * NW], iv[pr],
                                  isem[pr]).wait()

        def fire_data(m, i):
            pltpu.async_copy(g_hbm.at[pl.ds(_chunk_base(wid, m), GRP)],
                             buf[i], dsem[i])

        def wait_data(m, i):
            pltpu.make_async_copy(
                g_hbm.at[pl.ds(_chunk_base(wid, m), GRP)], buf[i],
                dsem[i]).wait()

        def fire_scat(m, i):
            r = m % 8
            kp = (m // 8) % 2

            @pl.when(kp == 0)
            def _():
                pltpu.async_copy(buf[i], table_sh.at[iv0.at[r]], ssem[i],
                                 add=True)

            @pl.when(kp == 1)
            def _():
                pltpu.async_copy(buf[i], table_sh.at[iv1.at[r]], ssem[i],
                                 add=True)

        def wait_scat(m, i):
            r = m % 8
            kp = (m // 8) % 2

            @pl.when(kp == 0)
            def _():
                pltpu.make_async_copy(buf[i], table_sh.at[iv0.at[r]],
                                      ssem[i]).wait()

            @pl.when(kp == 1)
            def _():
                pltpu.make_async_copy(buf[i], table_sh.at[iv1.at[r]],
                                      ssem[i]).wait()

        fire_idx(0, 0)
        fire_idx(1, 1)

        @pl.when(s < NS - 1)
        def _():
            pltpu.sync_copy(zero_hbm.at[pl.ds(s * RSUB, RSUB)],
                            table_sh.at[pl.ds(s * RSUB, RSUB)])

        @pl.when(s == NS - 1)
        def _():
            pltpu.sync_copy(zero_hbm.at[pl.ds(15 * RSUB, RLAST)],
                            table_sh.at[pl.ds(15 * RSUB, RLAST)])

        plsc.subcore_barrier()

        wait_idx(0, 0)
        fire_data(0, 0)
        wait_data(0, 0)
        fire_scat(0, 0)
        fire_data(1, 1)
        wait_data(1, 1)
        fire_scat(1, 1)
        wait_scat(0, 0)
        fire_data(2, 0)

        def body(q, _):
            m0 = 2 * q
            k = m0 // 8

            @pl.when(m0 % 8 == 0)
            def _():
                @pl.when(k % 2 == 0)
                def _():
                    wait_idx(k, 0)

                @pl.when(k % 2 == 1)
                def _():
                    wait_idx(k, 1)

            wait_data(m0, 0)
            fire_scat(m0, 0)
            wait_scat(m0 - 1, 1)

            @pl.when(m0 % 8 == 0)
            def _():
                @pl.when(k + 1 < niter)
                def _():
                    @pl.when((k + 1) % 2 == 0)
                    def _():
                        fire_idx(k + 1, 0)

                    @pl.when((k + 1) % 2 == 1)
                    def _():
                        fire_idx(k + 1, 1)

            fire_data(m0 + 1, 1)
            wait_data(m0 + 1, 1)
            fire_scat(m0 + 1, 1)
            wait_scat(m0, 0)

            @pl.when(m0 + 2 < M)
            def _():
                fire_data(m0 + 2, 0)

            return 0

        lax.fori_loop(1, M // 2, body, 0)
        wait_scat(M - 1, 1)
        plsc.subcore_barrier()

        @pl.when(s < NS - 1)
        def _():
            pltpu.sync_copy(table_sh.at[pl.ds(s * RSUB, RSUB)],
                            out_hbm.at[c, pl.ds(s * RSUB, RSUB)])

        @pl.when(s == NS - 1)
        def _():
            pltpu.sync_copy(table_sh.at[pl.ds(15 * RSUB, RLAST)],
                            out_hbm.at[c, pl.ds(15 * RSUB, RLAST)])

    return scatter_kernel


_sc_gather_a = _make_sc_gather(NSUP_A)
_sc_gather_b = _make_sc_gather(NSUP_B)
_sc_scatter_a = _make_sc_scatter(NSUP_A)
_sc_scatter_b = _make_sc_scatter(NSUP_B)


def _dot(a, b):
    return jnp.dot(a, b, preferred_element_type=jnp.float32)


def _tc_xw_body(x_ref, w_ref, o_ref):
    o_ref[...] = _dot(x_ref[...], w_ref[...])


def _tc_xw(x, w):
    return pl.pallas_call(
        _tc_xw_body,
        out_shape=jax.ShapeDtypeStruct((N, H), jnp.float32),
    )(x, w)


def _tc_add4_body(pa_ref, pb_ref, o_ref):
    o_ref[...] = (pa_ref[0] + pa_ref[1]) + (pb_ref[0] + pb_ref[1])


def _tc_add4(pa, pb):
    return pl.pallas_call(
        _tc_add4_body,
        out_shape=jax.ShapeDtypeStruct((N, H), jnp.float32),
    )(pa, pb)


def _tc_edge_init_body(xr_ref, ea_ref, we_ref, bei_ref, w0_ref,
                       h0_ref, g0_ref):
    h0 = jax.nn.relu(xr_ref[...] + _dot(ea_ref[...], we_ref[...])
                     + bei_ref[...])
    h0_ref[...] = h0
    g0_ref[...] = _dot(h0, w0_ref[...])


def _make_tc_edge_init(nrows, blk):
    def run(xr, ea, we, bei, w0):
        return pl.pallas_call(
            _tc_edge_init_body,
            grid=(nrows // blk,),
            in_specs=[
                pl.BlockSpec((blk, H), lambda i: (i, 0)),
                pl.BlockSpec((blk, DE), lambda i: (i, 0)),
                pl.BlockSpec((DE, H), lambda i: (0, 0)),
                pl.BlockSpec((1, H), lambda i: (0, 0)),
                pl.BlockSpec((H, H), lambda i: (0, 0)),
            ],
            out_specs=[
                pl.BlockSpec((blk, H), lambda i: (i, 0)),
                pl.BlockSpec((blk, H), lambda i: (i, 0)),
            ],
            out_shape=[
                jax.ShapeDtypeStruct((nrows, H), jnp.float32),
                jax.ShapeDtypeStruct((nrows, H), jnp.float32),
            ],
        )(xr, ea, we, bei, w0)
    return run


def _tc_conv_body(ar_ref, g_ref, h0_ref, w_ref, b_ref, o_ref):
    ge = g_ref[:, 0, :]
    go = g_ref[:, 1, :]
    he = jax.nn.relu(ar_ref[:, 0, :] - go + b_ref[...] + h0_ref[:, 0, :])
    ho = jax.nn.relu(ar_ref[:, 1, :] - ge + b_ref[...] + h0_ref[:, 1, :])
    o_ref[:, 0, :] = _dot(he, w_ref[...])
    o_ref[:, 1, :] = _dot(ho, w_ref[...])


def _tc_conv_last_body(ar_ref, g_ref, h0_ref, b_ref, o_ref):
    ge = g_ref[:, 0, :]
    go = g_ref[:, 1, :]
    o_ref[:, 0, :] = jax.nn.relu(ar_ref[:, 0, :] - go + b_ref[...]
                                 + h0_ref[:, 0, :])
    o_ref[:, 1, :] = jax.nn.relu(ar_ref[:, 1, :] - ge + b_ref[...]
                                 + h0_ref[:, 1, :])


def _make_tc_conv(npairs, blk):
    spec = pl.BlockSpec((blk, 2, H), lambda i: (i, 0, 0))

    def run(ar3, g3, h03, w, b):
        return pl.pallas_call(
            _tc_conv_body,
            grid=(npairs // blk,),
            in_specs=[
                spec, spec, spec,
                pl.BlockSpec((H, H), lambda i: (0, 0)),
                pl.BlockSpec((1, H), lambda i: (0, 0)),
            ],
            out_specs=spec,
            out_shape=jax.ShapeDtypeStruct((npairs, 2, H), jnp.float32),
        )(ar3, g3, h03, w, b)
    return run


def _make_tc_conv_last(npairs, blk):
    spec = pl.BlockSpec((blk, 2, H), lambda i: (i, 0, 0))

    def run(ar3, g3, h03, b):
        return pl.pallas_call(
            _tc_conv_last_body,
            grid=(npairs // blk,),
            in_specs=[
                spec, spec, spec,
                pl.BlockSpec((1, H), lambda i: (0, 0)),
            ],
            out_specs=spec,
            out_shape=jax.ShapeDtypeStruct((npairs, 2, H), jnp.float32),
        )(ar3, g3, h03, b)
    return run


_tc_edge_init_a = _make_tc_edge_init(E_A, 9984)
_tc_edge_init_b = _make_tc_edge_init(E_B, 10016)
_tc_conv_a = _make_tc_conv(E_A // 2, 4992)
_tc_conv_b = _make_tc_conv(E_B // 2, 5008)
_tc_conv_last_a = _make_tc_conv_last(E_A // 2, 4992)
_tc_conv_last_b = _make_tc_conv_last(E_B // 2, 5008)

_BN = 1000


def _tc_final_body(x_ref, s_ref, b2_ref, w1_ref, w2_ref, be_ref, wf_ref,
                   bf_ref, o_ref, acc_ref):
    i = pl.program_id(0)

    @pl.when(i == 0)
    def _():
        acc_ref[...] = jnp.zeros_like(acc_ref)

    hn = jax.nn.relu(_dot(x_ref[...], w1_ref[...])
                     + _dot(s_ref[...], w2_ref[...]) + be_ref[...])
    onehot = (b2_ref[...] == lax.broadcasted_iota(jnp.int32, (_BN, G), 1))
    acc_ref[...] += lax.dot_general(
        onehot.astype(jnp.float32), hn,
        (((0,), (0,)), ((), ())), preferred_element_type=jnp.float32)

    @pl.when(i == pl.num_programs(0) - 1)
    def _():
        o_ref[...] = (jnp.sum(acc_ref[...] * wf_ref[...], axis=1,
                              keepdims=True) + bf_ref[...])


def _tc_final(x, s, batch2, w1, w2, be, wf_row, bf):
    return pl.pallas_call(
        _tc_final_body,
        grid=(N // _BN,),
        in_specs=[
            pl.BlockSpec((_BN, DN), lambda i: (i, 0)),
            pl.BlockSpec((_BN, H), lambda i: (i, 0)),
            pl.BlockSpec((_BN, 1), lambda i: (i, 0)),
            pl.BlockSpec((DN, H), lambda i: (0, 0)),
            pl.BlockSpec((H, H), lambda i: (0, 0)),
            pl.BlockSpec((1, H), lambda i: (0, 0)),
            pl.BlockSpec((1, H), lambda i: (0, 0)),
            pl.BlockSpec((1, 1), lambda i: (0, 0)),
        ],
        out_specs=pl.BlockSpec((G, 1), lambda i: (0, 0)),
        out_shape=jax.ShapeDtypeStruct((G, 1), jnp.float32),
        scratch_shapes=[pltpu.VMEM((G, H), jnp.float32)],
    )(x, s, batch2, w1, w2, be, wf_row, bf)


def kernel(x, edge_attr, W_edge_init, b_edge_init, W_conv0, b_conv0,
           W_conv1, b_conv1, W_conv2, b_conv2, W_e2n, b_e2n, W_ffn, b_ffn,
           edge_index, batch):
    row = edge_index[0].astype(jnp.int32)
    col = edge_index[1].astype(jnp.int32)
    row3 = row.reshape(NSUP, 8, GRP)
    col3 = col.reshape(NSUP, 8, GRP)
    rowA, rowB = row3[:NSUP_A], row3[NSUP_A:]
    colA, colB = col3[:NSUP_A], col3[NSUP_A:]
    eaA, eaB = edge_attr[:E_A], edge_attr[E_A:]
    zeros_n = jnp.zeros((N, H), jnp.float32)

    bei = b_edge_init.reshape(1, H)
    b0 = b_conv0.reshape(1, H)
    b1 = b_conv1.reshape(1, H)
    b2 = b_conv2.reshape(1, H)
    be = b_e2n.reshape(1, H)
    wf_row = W_ffn.reshape(1, H)
    bf = b_ffn.reshape(1, 1)
    batch2 = batch.astype(jnp.int32).reshape(N, 1)

    xw = _tc_xw(x, W_edge_init[:DN])
    xrA = _sc_gather_a(xw, rowA)
    xrB = _sc_gather_b(xw, rowB)
    h0A, g0A = _tc_edge_init_a(xrA, eaA, W_edge_init[DN:], bei, W_conv0)
    h0B, g0B = _tc_edge_init_b(xrB, eaB, W_edge_init[DN:], bei, W_conv0)

    h0A3 = h0A.reshape(E_A // 2, 2, H)
    h0B3 = h0B.reshape(E_B // 2, 2, H)
    gA, gB = g0A, g0B
    for w_next, b_cur in ((W_conv1, b0), (W_conv2, b1)):
        pA = _sc_scatter_a(gA, colA, zeros_n)
        pB = _sc_scatter_b(gB, colB, zeros_n)
        a = _tc_add4(pA, pB)
        arA3 = _sc_gather_a(a, rowA).reshape(E_A // 2, 2, H)
        gA = _tc_conv_a(arA3, gA.reshape(E_A // 2, 2, H), h0A3,
                        w_next, b_cur).reshape(E_A, H)
        arB3 = _sc_gather_b(a, rowB).reshape(E_B // 2, 2, H)
        gB = _tc_conv_b(arB3, gB.reshape(E_B // 2, 2, H), h0B3,
                        w_next, b_cur).reshape(E_B, H)

    pA = _sc_scatter_a(gA, colA, zeros_n)
    pB = _sc_scatter_b(gB, colB, zeros_n)
    a = _tc_add4(pA, pB)
    arA3 = _sc_gather_a(a, rowA).reshape(E_A // 2, 2, H)
    h3A = _tc_conv_last_a(arA3, gA.reshape(E_A // 2, 2, H), h0A3, b2)
    arB3 = _sc_gather_b(a, rowB).reshape(E_B // 2, 2, H)
    h3B = _tc_conv_last_b(arB3, gB.reshape(E_B // 2, 2, H), h0B3, b2)

    pA = _sc_scatter_a(h3A.reshape(E_A, H), colA, zeros_n)
    pB = _sc_scatter_b(h3B.reshape(E_B, H), colB, zeros_n)
    s = _tc_add4(pA, pB)
    out = _tc_final(x, s, batch2, W_e2n[:DN], W_e2n[DN:], be, wf_row, bf)
    return out.reshape(G)
